# speedup vs baseline: 2.7442x; 2.7442x over previous
import functools

import jax
import jax.numpy as jnp
from jax import lax
from jax.experimental import pallas as pl
from jax.experimental.pallas import tpu as pltpu
from jax.experimental.pallas import tpu_sc as plsc

_D = 128
_NW = 32
_CHUNK = 128
_NCH = 50


def _sc_gather(idx3, weight):
    total = _NW * _NCH * _CHUNK
    mesh = plsc.VectorSubcoreMesh(core_axis_name="c", subcore_axis_name="s")

    @functools.partial(
        pl.kernel,
        out_type=jax.ShapeDtypeStruct((total, _D), jnp.float32),
        mesh=mesh,
        scratch_types=[
            pltpu.VMEM((_NCH, _CHUNK), jnp.int32),
            pltpu.VMEM((_CHUNK, _D), jnp.float32),
            pltpu.SemaphoreType.DMA,
        ],
    )
    def k(idx_hbm, w_hbm, out_hbm, idx_v, rows_v, sem):
        wid = lax.axis_index("s") * 2 + lax.axis_index("c")
        base = wid * (_NCH * _CHUNK)
        pltpu.sync_copy(idx_hbm.at[wid], idx_v)

        @pl.loop(0, _NCH)
        def _chunk(c):
            pltpu.async_copy(w_hbm.at[idx_v.at[c]], rows_v, sem).wait()
            pltpu.sync_copy(rows_v, out_hbm.at[pl.ds(base + c * _CHUNK, _CHUNK)])

    return k(idx3, weight)


def kernel(token_ids, weight):
    b, s = token_ids.shape
    idx3 = token_ids.reshape(_NW, _NCH, _CHUNK).astype(jnp.int32)
    out = _sc_gather(idx3, weight)
    return out.reshape(b, s, _D)


# speedup vs baseline: 3.0768x; 1.1212x over previous
import functools

import jax
import jax.numpy as jnp
from jax import lax
from jax.experimental import pallas as pl
from jax.experimental.pallas import tpu as pltpu
from jax.experimental.pallas import tpu_sc as plsc

_D = 128
_NW = 32
_CHUNK = 128
_NCH = 50
_NBUF = 5
_GLEAD = 3
_WLAG = _NBUF - _GLEAD


def _sc_gather(idx3, weight):
    total = _NW * _NCH * _CHUNK
    mesh = plsc.VectorSubcoreMesh(core_axis_name="c", subcore_axis_name="s")

    @functools.partial(
        pl.kernel,
        out_type=jax.ShapeDtypeStruct((total, _D), jnp.float32),
        mesh=mesh,
        scratch_types=(
            [pltpu.VMEM((_NCH, _CHUNK), jnp.int32),
             pltpu.VMEM((_NBUF, _CHUNK, _D), jnp.float32)]
            + [pltpu.SemaphoreType.DMA] * (2 * _NBUF)
        ),
    )
    def k(idx_hbm, w_hbm, out_hbm, idx_v, rows_v, *sems):
        gsem, wsem = sems[:_NBUF], sems[_NBUF:]
        wid = lax.axis_index("s") * 2 + lax.axis_index("c")
        base = wid * (_NCH * _CHUNK)
        pltpu.sync_copy(idx_hbm.at[wid], idx_v)

        def gather_start(c, b):
            pltpu.async_copy(w_hbm.at[idx_v.at[c]], rows_v.at[b], gsem[b])

        def gather_wait(b):
            pltpu.make_async_copy(
                w_hbm.at[idx_v.at[0]], rows_v.at[b], gsem[b]).wait()

        def write_start(c, b):
            pltpu.async_copy(
                rows_v.at[b], out_hbm.at[pl.ds(base + c * _CHUNK, _CHUNK)],
                wsem[b])

        def write_wait(b):
            pltpu.make_async_copy(
                rows_v.at[b], out_hbm.at[pl.ds(base, _CHUNK)], wsem[b]).wait()

        def step(c, b, do_wwait, do_gstart):
            gather_wait(b)
            write_start(c, b)
            if do_wwait:
                write_wait((b - _WLAG) % _NBUF)
            if do_gstart:
                gather_start(c + _GLEAD, (b + _GLEAD) % _NBUF)

        for c in range(_GLEAD):
            gather_start(c, c % _NBUF)
        for c in range(_WLAG):
            step(c, c % _NBUF, do_wwait=False, do_gstart=True)

        @pl.loop(_WLAG, _NCH - _GLEAD, step=_NBUF)
        def _grp(g):
            for j in range(_NBUF):
                step(g + j, (_WLAG + j) % _NBUF, do_wwait=True, do_gstart=True)

        for c in range(_NCH - _GLEAD, _NCH):
            step(c, c % _NBUF, do_wwait=True, do_gstart=False)
        for c in range(_NCH - _WLAG, _NCH):
            write_wait(c % _NBUF)

    return k(idx3, weight)


def kernel(token_ids, weight):
    b, s = token_ids.shape
    idx3 = token_ids.reshape(_NW, _NCH, _CHUNK).astype(jnp.int32)
    out = _sc_gather(idx3, weight)
    return out.reshape(b, s, _D)


# speedup vs baseline: 5.5156x; 1.7926x over previous
import functools

import jax
import jax.numpy as jnp
from jax import lax
from jax.experimental import pallas as pl
from jax.experimental.pallas import tpu as pltpu
from jax.experimental.pallas import tpu_sc as plsc

_D = 128
_S = 50
_NW = 32
_EPC = 2
_CHUNK = _EPC * _S
_NCH = 64
_NBUF = 8
_GLEAD = 5
_WLAG = _NBUF - _GLEAD


def _sc_gather(idx3, weight, n_batch):
    mesh = plsc.VectorSubcoreMesh(core_axis_name="c", subcore_axis_name="s")

    @functools.partial(
        pl.kernel,
        out_type=jax.ShapeDtypeStruct((n_batch, _S, _D), jnp.float32),
        mesh=mesh,
        scratch_types=(
            [pltpu.VMEM((_NCH, _CHUNK), jnp.int32),
             pltpu.VMEM((_NBUF, _CHUNK, _D), jnp.float32)]
            + [pltpu.SemaphoreType.DMA] * (2 * _NBUF)
        ),
    )
    def k(idx_hbm, w_hbm, out_hbm, idx_v, rows_v, *sems):
        gsem, wsem = sems[:_NBUF], sems[_NBUF:]
        wid = lax.axis_index("s") * 2 + lax.axis_index("c")
        base = wid * (_NCH * _EPC)
        pltpu.sync_copy(idx_hbm.at[wid], idx_v)

        def gather_start(c, b):
            pltpu.async_copy(w_hbm.at[idx_v.at[c]], rows_v.at[b], gsem[b])

        def gather_wait(b):
            pltpu.make_async_copy(
                w_hbm.at[idx_v.at[0]], rows_v.at[b], gsem[b]).wait()

        def write_start(c, b):
            pltpu.async_copy(
                rows_v.at[b].reshape(_EPC, _S, _D),
                out_hbm.at[pl.ds(base + c * _EPC, _EPC)], wsem[b])

        def write_wait(b):
            pltpu.make_async_copy(
                rows_v.at[b].reshape(_EPC, _S, _D),
                out_hbm.at[pl.ds(base, _EPC)], wsem[b]).wait()

        def step(c, b, do_wwait, do_gstart):
            gather_wait(b)
            write_start(c, b)
            if do_wwait:
                write_wait((b - _WLAG) % _NBUF)
            if do_gstart:
                gather_start(c + _GLEAD, (b + _GLEAD) % _NBUF)

        for c in range(_GLEAD):
            gather_start(c, c % _NBUF)
        for c in range(_WLAG):
            step(c, c % _NBUF, do_wwait=False, do_gstart=True)

        @pl.loop(_WLAG, _NCH - _GLEAD, step=_NBUF)
        def _grp(g):
            for j in range(_NBUF):
                step(g + j, (_WLAG + j) % _NBUF, do_wwait=True, do_gstart=True)

        for c in range(_NCH - _GLEAD, _NCH):
            step(c, c % _NBUF, do_wwait=True, do_gstart=False)
        for c in range(_NCH - _WLAG, _NCH):
            write_wait(c % _NBUF)

    return k(idx3, weight)


def kernel(token_ids, weight):
    n_batch, s = token_ids.shape
    idx3 = token_ids.reshape(_NW, _NCH, _CHUNK).astype(jnp.int32)
    return _sc_gather(idx3, weight, n_batch)


# speedup vs baseline: 5.5205x; 1.0009x over previous
import functools

import jax
import jax.numpy as jnp
from jax import lax
from jax.experimental import pallas as pl
from jax.experimental.pallas import tpu as pltpu
from jax.experimental.pallas import tpu_sc as plsc

_D = 128
_S = 50
_NW = 32
_EPC = 2
_CHUNK = _EPC * _S
_NCH = 64
_NBUF = 8
_GLEAD = 5
_WLAG = _NBUF - _GLEAD


def _sc_gather(idx3, weight, n_batch):
    mesh = plsc.VectorSubcoreMesh(core_axis_name="c", subcore_axis_name="s")

    @functools.partial(
        pl.kernel,
        out_type=jax.ShapeDtypeStruct((n_batch, _S, _D), jnp.float32),
        mesh=mesh,
        scratch_types=(
            [pltpu.VMEM((_NCH, _CHUNK), jnp.int32),
             pltpu.VMEM((_NBUF, _CHUNK, _D), jnp.float32)]
            + [pltpu.SemaphoreType.DMA] * (2 * _NBUF)
        ),
        compiler_params=pltpu.CompilerParams(use_tc_tiling_on_sc=True),
    )
    def k(idx_hbm, w_hbm, out_hbm, idx_v, rows_v, *sems):
        gsem, wsem = sems[:_NBUF], sems[_NBUF:]
        wid = lax.axis_index("s") * 2 + lax.axis_index("c")
        base = wid * (_NCH * _EPC)
        pltpu.sync_copy(idx_hbm.at[wid], idx_v)

        def gather_start(c, b):
            pltpu.async_copy(w_hbm.at[idx_v.at[c]], rows_v.at[b], gsem[b])

        def gather_wait(b):
            pltpu.make_async_copy(
                w_hbm.at[idx_v.at[0]], rows_v.at[b], gsem[b]).wait()

        def write_start(c, b):
            pltpu.async_copy(
                rows_v.at[b].reshape(_EPC, _S, _D),
                out_hbm.at[pl.ds(base + c * _EPC, _EPC)], wsem[b])

        def write_wait(b):
            pltpu.make_async_copy(
                rows_v.at[b].reshape(_EPC, _S, _D),
                out_hbm.at[pl.ds(base, _EPC)], wsem[b]).wait()

        def step(c, b, do_wwait, do_gstart):
            gather_wait(b)
            write_start(c, b)
            if do_wwait:
                write_wait((b - _WLAG) % _NBUF)
            if do_gstart:
                gather_start(c + _GLEAD, (b + _GLEAD) % _NBUF)

        for c in range(_GLEAD):
            gather_start(c, c % _NBUF)
        for c in range(_WLAG):
            step(c, c % _NBUF, do_wwait=False, do_gstart=True)

        @pl.loop(_WLAG, _NCH - _GLEAD, step=_NBUF)
        def _grp(g):
            for j in range(_NBUF):
                step(g + j, (_WLAG + j) % _NBUF, do_wwait=True, do_gstart=True)

        for c in range(_NCH - _GLEAD, _NCH):
            step(c, c % _NBUF, do_wwait=True, do_gstart=False)
        for c in range(_NCH - _WLAG, _NCH):
            write_wait(c % _NBUF)

    return k(idx3, weight)


def kernel(token_ids, weight):
    n_batch, s = token_ids.shape
    idx3 = token_ids.reshape(_NW, _NCH, _CHUNK).astype(jnp.int32)
    return _sc_gather(idx3, weight, n_batch)


# speedup vs baseline: 9.9027x; 1.7938x over previous
import functools

import jax
import jax.numpy as jnp
from jax import lax
from jax.experimental import pallas as pl
from jax.experimental.pallas import tpu as pltpu
from jax.experimental.pallas import tpu_sc as plsc

_D = 128
_NW = 32
_CHUNK = 128
_NCH = 50
_NBUF = 5
_GLEAD = 3
_WLAG = _NBUF - _GLEAD


def _sc_gather(idx_t, weight, n_batch):
    mesh = plsc.VectorSubcoreMesh(core_axis_name="c", subcore_axis_name="s")

    @functools.partial(
        pl.kernel,
        out_type=jax.ShapeDtypeStruct((_NCH, n_batch, _D), jnp.float32),
        mesh=mesh,
        scratch_types=(
            [pltpu.VMEM((_NCH, _CHUNK), jnp.int32),
             pltpu.VMEM((_NBUF, _CHUNK, _D), jnp.float32)]
            + [pltpu.SemaphoreType.DMA] * (2 * _NBUF)
        ),
    )
    def k(idx_hbm, w_hbm, out_hbm, idx_v, rows_v, *sems):
        gsem, wsem = sems[:_NBUF], sems[_NBUF:]
        wid = lax.axis_index("s") * 2 + lax.axis_index("c")
        col0 = wid * _CHUNK
        pltpu.sync_copy(idx_hbm.at[:, pl.ds(col0, _CHUNK)], idx_v)

        def gather_start(c, b):
            pltpu.async_copy(w_hbm.at[idx_v.at[c]], rows_v.at[b], gsem[b])

        def gather_wait(b):
            pltpu.make_async_copy(
                w_hbm.at[idx_v.at[0]], rows_v.at[b], gsem[b]).wait()

        def write_start(c, b):
            pltpu.async_copy(
                rows_v.at[b], out_hbm.at[c, pl.ds(col0, _CHUNK)], wsem[b])

        def write_wait(b):
            pltpu.make_async_copy(
                rows_v.at[b], out_hbm.at[0, pl.ds(col0, _CHUNK)],
                wsem[b]).wait()

        def step(c, b, do_wwait, do_gstart):
            gather_wait(b)
            write_start(c, b)
            if do_wwait:
                write_wait((b - _WLAG) % _NBUF)
            if do_gstart:
                gather_start(c + _GLEAD, (b + _GLEAD) % _NBUF)

        for c in range(_GLEAD):
            gather_start(c, c % _NBUF)
        for c in range(_WLAG):
            step(c, c % _NBUF, do_wwait=False, do_gstart=True)

        @pl.loop(_WLAG, _NCH - _GLEAD, step=_NBUF)
        def _grp(g):
            for j in range(_NBUF):
                step(g + j, (_WLAG + j) % _NBUF, do_wwait=True, do_gstart=True)

        for c in range(_NCH - _GLEAD, _NCH):
            step(c, c % _NBUF, do_wwait=True, do_gstart=False)
        for c in range(_NCH - _WLAG, _NCH):
            write_wait(c % _NBUF)

    return k(idx_t, weight)


def kernel(token_ids, weight):
    n_batch, s = token_ids.shape
    idx_t = token_ids.T.astype(jnp.int32)
    out = _sc_gather(idx_t, weight, n_batch)
    return jnp.transpose(out, (1, 0, 2))
